# baseline (speedup 1.0000x reference)
import functools

import jax
import jax.numpy as jnp
from jax import lax
from jax.experimental import pallas as pl
from jax.experimental.pallas import tpu as pltpu
from jax.experimental.pallas import tpu_sc as plsc

F = 26
VOCAB = 100000
D = 16
B = 4096
H1 = 32
H2 = 32

NC, NS = 2, 16
NW = NC * NS
ROWS = B * F
RPW = ROWS // NW
CH = 128
NCH = RPW // CH

_sc_mesh = plsc.VectorSubcoreMesh(core_axis_name="c", subcore_axis_name="s",
                                  num_cores=NC, num_subcores=NS)


@functools.partial(
    pl.kernel,
    out_type=(
        jax.ShapeDtypeStruct((ROWS, D), jnp.bfloat16),
        jax.ShapeDtypeStruct((ROWS,), jnp.float32),
    ),
    mesh=_sc_mesh,
    scratch_types=[
        pltpu.VMEM((NCH, CH), jnp.int32),
        pltpu.VMEM((RPW, D), jnp.bfloat16),
        pltpu.VMEM((RPW,), jnp.float32),
        pltpu.SemaphoreType.DMA,
        pltpu.SemaphoreType.DMA,
    ],
    compiler_params=pltpu.CompilerParams(use_tc_tiling_on_sc=False),
)
def _sc_gather(w2_hbm, w1_hbm, idx_hbm, e2_out, fm1_out,
               idx_v, rows_v, fm1_v, sem2, sem1):
    wid = lax.axis_index("s") * NC + lax.axis_index("c")
    base = wid * RPW
    pltpu.sync_copy(idx_hbm.at[wid], idx_v)
    cps = []
    for j in range(NCH):
        cps.append(pltpu.async_copy(
            w2_hbm.at[idx_v.at[j]], rows_v.at[pl.ds(j * CH, CH)], sem2))
        cps.append(pltpu.async_copy(
            w1_hbm.at[idx_v.at[j]], fm1_v.at[pl.ds(j * CH, CH)], sem1))
    for cp in cps:
        cp.wait()
    pltpu.sync_copy(rows_v, e2_out.at[pl.ds(base, RPW)])
    pltpu.sync_copy(fm1_v, fm1_out.at[pl.ds(base, RPW)])


BB = 1024


def _tc_body(e2_ref, fm1_ref, xv_ref, wl1_ref, bl1_ref, wl2_ref, bl2_ref,
             bias_ref, out_ref):
    E = e2_ref[...].astype(jnp.float32)
    V = xv_ref[...]
    hp = jax.lax.Precision.HIGHEST
    R = (lax.broadcasted_iota(jnp.int32, (F, F * D), 1) // D
         == lax.broadcasted_iota(jnp.int32, (F, F * D), 0)).astype(jnp.float32)
    Es = E * jax.lax.dot(V, R, precision=hp)
    S = (lax.broadcasted_iota(jnp.int32, (F * D, D), 0) % D
         == lax.broadcasted_iota(jnp.int32, (F * D, D), 1)).astype(jnp.float32)
    sum_emb = jax.lax.dot(Es, S, precision=hp)
    sum_sq = jax.lax.dot(Es * Es, S, precision=hp)
    fm2 = 0.5 * (sum_emb * sum_emb - sum_sq)
    h = jnp.maximum(jax.lax.dot(Es, wl1_ref[...], precision=hp)
                    + bl1_ref[...], 0.0)
    h = jnp.maximum(jax.lax.dot(h, wl2_ref[...], precision=hp)
                    + bl2_ref[...], 0.0)
    total = (jnp.sum(fm1_ref[...] * V, axis=1, keepdims=True)
             + jnp.sum(fm2, axis=1, keepdims=True)
             + jnp.sum(h, axis=1, keepdims=True)
             + bias_ref[...])
    out_ref[...] = total


_tc_dense = pl.pallas_call(
    _tc_body,
    grid=(B // BB,),
    in_specs=[
        pl.BlockSpec((BB, F * D), lambda i: (i, 0)),
        pl.BlockSpec((BB, F), lambda i: (i, 0)),
        pl.BlockSpec((BB, F), lambda i: (i, 0)),
        pl.BlockSpec((F * D, H1), lambda i: (0, 0)),
        pl.BlockSpec((1, H1), lambda i: (0, 0)),
        pl.BlockSpec((H1, H2), lambda i: (0, 0)),
        pl.BlockSpec((1, H2), lambda i: (0, 0)),
        pl.BlockSpec((1, 1), lambda i: (0, 0)),
    ],
    out_specs=pl.BlockSpec((BB, 1), lambda i: (i, 0)),
    out_shape=jax.ShapeDtypeStruct((B, 1), jnp.float32),
)


def kernel(Xi, Xv, W1, W2, Wl1, bl1, Wl2, bl2, bias):
    Xi_s = Xi[:, :, 0].astype(jnp.int32)
    flat_idx = (Xi_s + jnp.arange(F, dtype=jnp.int32)[None, :] * VOCAB)
    idx3d = flat_idx.reshape(NW, NCH, CH)
    w2_flat = W2.astype(jnp.bfloat16).reshape(F * VOCAB, D)
    w1_flat = W1.reshape(F * VOCAB)
    e2_rows, fm1_flat = _sc_gather(w2_flat, w1_flat, idx3d)
    out = _tc_dense(e2_rows.reshape(B, F * D), fm1_flat.reshape(B, F), Xv,
                    Wl1, bl1.reshape(1, H1), Wl2, bl2.reshape(1, H2),
                    bias.reshape(1, 1))
    return out.reshape(B)


# speedup vs baseline: 1.1854x; 1.1854x over previous
import functools

import jax
import jax.numpy as jnp
from jax import lax
from jax.experimental import pallas as pl
from jax.experimental.pallas import tpu as pltpu
from jax.experimental.pallas import tpu_sc as plsc

F = 26
VOCAB = 100000
D = 16
B = 4096
H1 = 32
H2 = 32

NC, NS = 2, 16
NW = NC * NS
ROWS = B * F
RPW = ROWS // NW
CH = 128
NCH = RPW // CH

_sc_mesh = plsc.VectorSubcoreMesh(core_axis_name="c", subcore_axis_name="s",
                                  num_cores=NC, num_subcores=NS)


@functools.partial(
    pl.kernel,
    out_type=(
        jax.ShapeDtypeStruct((ROWS, D), jnp.float32),
        jax.ShapeDtypeStruct((ROWS,), jnp.float32),
    ),
    mesh=_sc_mesh,
    scratch_types=[
        pltpu.VMEM((NCH, CH), jnp.int32),
        pltpu.VMEM((RPW, D), jnp.float32),
        pltpu.VMEM((RPW,), jnp.float32),
        pltpu.SemaphoreType.DMA,
        pltpu.SemaphoreType.DMA,
    ],
    compiler_params=pltpu.CompilerParams(use_tc_tiling_on_sc=False),
)
def _sc_gather(w2_hbm, w1_hbm, idx_hbm, e2_out, fm1_out,
               idx_v, rows_v, fm1_v, sem2, sem1):
    wid = lax.axis_index("s") * NC + lax.axis_index("c")
    base = wid * RPW
    pltpu.sync_copy(idx_hbm.at[wid], idx_v)
    cps = []
    for j in range(NCH):
        cps.append(pltpu.async_copy(
            w2_hbm.at[idx_v.at[j]], rows_v.at[pl.ds(j * CH, CH)], sem2))
        cps.append(pltpu.async_copy(
            w1_hbm.at[idx_v.at[j]], fm1_v.at[pl.ds(j * CH, CH)], sem1))
    for cp in cps:
        cp.wait()
    pltpu.sync_copy(rows_v, e2_out.at[pl.ds(base, RPW)])
    pltpu.sync_copy(fm1_v, fm1_out.at[pl.ds(base, RPW)])


BB = 1024


def _tc_body(e2_ref, fm1_ref, xv_ref, wl1_ref, bl1_ref, wl2_ref, bl2_ref,
             bias_ref, out_ref):
    E = e2_ref[...]
    V = xv_ref[...]
    hp = jax.lax.Precision.HIGHEST
    R = (lax.broadcasted_iota(jnp.int32, (F, F * D), 1) // D
         == lax.broadcasted_iota(jnp.int32, (F, F * D), 0)).astype(jnp.float32)
    Es = E * jax.lax.dot(V, R, precision=hp)
    S = (lax.broadcasted_iota(jnp.int32, (F * D, D), 0) % D
         == lax.broadcasted_iota(jnp.int32, (F * D, D), 1)).astype(jnp.float32)
    sum_emb = jax.lax.dot(Es, S, precision=hp)
    sum_sq = jax.lax.dot(Es * Es, S, precision=hp)
    fm2 = 0.5 * (sum_emb * sum_emb - sum_sq)
    h = jnp.maximum(jax.lax.dot(Es, wl1_ref[...], precision=hp)
                    + bl1_ref[...], 0.0)
    h = jnp.maximum(jax.lax.dot(h, wl2_ref[...], precision=hp)
                    + bl2_ref[...], 0.0)
    total = (jnp.sum(fm1_ref[...] * V, axis=1, keepdims=True)
             + jnp.sum(fm2, axis=1, keepdims=True)
             + jnp.sum(h, axis=1, keepdims=True)
             + bias_ref[...])
    out_ref[...] = total


_tc_dense = pl.pallas_call(
    _tc_body,
    grid=(B // BB,),
    in_specs=[
        pl.BlockSpec((BB, F * D), lambda i: (i, 0)),
        pl.BlockSpec((BB, F), lambda i: (i, 0)),
        pl.BlockSpec((BB, F), lambda i: (i, 0)),
        pl.BlockSpec((F * D, H1), lambda i: (0, 0)),
        pl.BlockSpec((1, H1), lambda i: (0, 0)),
        pl.BlockSpec((H1, H2), lambda i: (0, 0)),
        pl.BlockSpec((1, H2), lambda i: (0, 0)),
        pl.BlockSpec((1, 1), lambda i: (0, 0)),
    ],
    out_specs=pl.BlockSpec((BB, 1), lambda i: (i, 0)),
    out_shape=jax.ShapeDtypeStruct((B, 1), jnp.float32),
)


def kernel(Xi, Xv, W1, W2, Wl1, bl1, Wl2, bl2, bias):
    Xi_s = Xi[:, :, 0].astype(jnp.int32)
    flat_idx = (Xi_s + jnp.arange(F, dtype=jnp.int32)[None, :] * VOCAB)
    idx3d = flat_idx.reshape(NW, NCH, CH)
    w2_flat = W2.reshape(F * VOCAB, D)
    w1_flat = W1.reshape(F * VOCAB)
    e2_rows, fm1_flat = _sc_gather(w2_flat, w1_flat, idx3d)
    out = _tc_dense(e2_rows.reshape(B, F * D), fm1_flat.reshape(B, F), Xv,
                    Wl1, bl1.reshape(1, H1), Wl2, bl2.reshape(1, H2),
                    bias.reshape(1, 1))
    return out.reshape(B)


# speedup vs baseline: 2.3267x; 1.9628x over previous
import functools

import jax
import jax.numpy as jnp
from jax import lax
from jax.experimental import pallas as pl
from jax.experimental.pallas import tpu as pltpu
from jax.experimental.pallas import tpu_sc as plsc

F = 26
VOCAB = 100000
D = 16
B = 4096
H1 = 32
H2 = 32

NC, NS = 2, 16
NW = NC * NS
ROWS = B * F
RPW = ROWS // NW
CH = 128
NCH = RPW // CH

_sc_mesh = plsc.VectorSubcoreMesh(core_axis_name="c", subcore_axis_name="s",
                                  num_cores=NC, num_subcores=NS)


NCHE = RPW * D // CH


@functools.partial(
    pl.kernel,
    out_type=(
        jax.ShapeDtypeStruct((ROWS * D,), jnp.float32),
        jax.ShapeDtypeStruct((ROWS,), jnp.float32),
    ),
    mesh=_sc_mesh,
    scratch_types=[
        pltpu.VMEM((NCHE, CH), jnp.int32),
        pltpu.VMEM((NCH, CH), jnp.int32),
        pltpu.VMEM((RPW * D,), jnp.float32),
        pltpu.VMEM((RPW,), jnp.float32),
        pltpu.SemaphoreType.DMA,
        pltpu.SemaphoreType.DMA,
    ],
    compiler_params=pltpu.CompilerParams(use_tc_tiling_on_sc=False),
)
def _sc_gather(w2_hbm, w1_hbm, eidx_hbm, idx_hbm, e2_out, fm1_out,
               eidx_v, idx_v, rows_v, fm1_v, sem2, sem1):
    wid = lax.axis_index("s") * NC + lax.axis_index("c")
    base = wid * RPW
    pltpu.sync_copy(eidx_hbm.at[wid], eidx_v)
    pltpu.sync_copy(idx_hbm.at[wid], idx_v)

    def fire(c, _):
        pltpu.async_copy(w2_hbm.at[eidx_v.at[c]],
                         rows_v.at[pl.ds(c * CH, CH)], sem2)
        return ()

    lax.fori_loop(0, NCHE, fire, ())
    cps = []
    for j in range(NCH):
        cps.append(pltpu.async_copy(
            w1_hbm.at[idx_v.at[j]], fm1_v.at[pl.ds(j * CH, CH)], sem1))

    def drain(c, _):
        pltpu.make_async_copy(w2_hbm.at[eidx_v.at[0]],
                              rows_v.at[pl.ds(0, CH)], sem2).wait()
        return ()

    lax.fori_loop(0, NCHE, drain, ())
    for cp in cps:
        cp.wait()
    pltpu.sync_copy(rows_v, e2_out.at[pl.ds(base * D, RPW * D)])
    pltpu.sync_copy(fm1_v, fm1_out.at[pl.ds(base, RPW)])


BB = 1024


def _tc_body(e2_ref, fm1_ref, xv_ref, wl1_ref, bl1_ref, wl2_ref, bl2_ref,
             bias_ref, out_ref):
    E = e2_ref[...]
    V = xv_ref[...]
    hp = jax.lax.Precision.HIGHEST
    R = (lax.broadcasted_iota(jnp.int32, (F, F * D), 1) // D
         == lax.broadcasted_iota(jnp.int32, (F, F * D), 0)).astype(jnp.float32)
    Es = E * jax.lax.dot(V, R, precision=hp)
    S = (lax.broadcasted_iota(jnp.int32, (F * D, D), 0) % D
         == lax.broadcasted_iota(jnp.int32, (F * D, D), 1)).astype(jnp.float32)
    sum_emb = jax.lax.dot(Es, S, precision=hp)
    sum_sq = jax.lax.dot(Es * Es, S, precision=hp)
    fm2 = 0.5 * (sum_emb * sum_emb - sum_sq)
    h = jnp.maximum(jax.lax.dot(Es, wl1_ref[...], precision=hp)
                    + bl1_ref[...], 0.0)
    h = jnp.maximum(jax.lax.dot(h, wl2_ref[...], precision=hp)
                    + bl2_ref[...], 0.0)
    total = (jnp.sum(fm1_ref[...] * V, axis=1, keepdims=True)
             + jnp.sum(fm2, axis=1, keepdims=True)
             + jnp.sum(h, axis=1, keepdims=True)
             + bias_ref[...])
    out_ref[...] = total


_tc_dense = pl.pallas_call(
    _tc_body,
    grid=(B // BB,),
    in_specs=[
        pl.BlockSpec((BB, F * D), lambda i: (i, 0)),
        pl.BlockSpec((BB, F), lambda i: (i, 0)),
        pl.BlockSpec((BB, F), lambda i: (i, 0)),
        pl.BlockSpec((F * D, H1), lambda i: (0, 0)),
        pl.BlockSpec((1, H1), lambda i: (0, 0)),
        pl.BlockSpec((H1, H2), lambda i: (0, 0)),
        pl.BlockSpec((1, H2), lambda i: (0, 0)),
        pl.BlockSpec((1, 1), lambda i: (0, 0)),
    ],
    out_specs=pl.BlockSpec((BB, 1), lambda i: (i, 0)),
    out_shape=jax.ShapeDtypeStruct((B, 1), jnp.float32),
)


def kernel(Xi, Xv, W1, W2, Wl1, bl1, Wl2, bl2, bias):
    Xi_s = Xi[:, :, 0].astype(jnp.int32)
    flat_idx = (Xi_s + jnp.arange(F, dtype=jnp.int32)[None, :] * VOCAB)
    idx3d = flat_idx.reshape(NW, NCH, CH)
    eidx = (Xi_s[:, :, None]
            + (jnp.arange(F * D, dtype=jnp.int32) * VOCAB).reshape(F, D)[None])
    eidx3d = eidx.reshape(NW, NCHE, CH)
    w2_flat = W2.transpose(0, 2, 1).reshape(F * VOCAB * D)
    w1_flat = W1.reshape(F * VOCAB)
    e2_elems, fm1_flat = _sc_gather(w2_flat, w1_flat, eidx3d, idx3d)
    out = _tc_dense(e2_elems.reshape(B, F * D), fm1_flat.reshape(B, F), Xv,
                    Wl1, bl1.reshape(1, H1), Wl2, bl2.reshape(1, H2),
                    bias.reshape(1, 1))
    return out.reshape(B)
